# Initial kernel scaffold; baseline (speedup 1.0000x reference)
import jax
import jax.numpy as jnp
from jax.experimental import pallas as pl


def kernel(x, lengths, W0, b0, W1, b1, W2, b2):
    raise NotImplementedError("write your pallas kernel here")


# speedup vs baseline: 53.5096x; 53.5096x over previous
import math

import jax
import jax.numpy as jnp
from jax.experimental import pallas as pl
from jax.experimental.pallas import tpu as pltpu

_HD = 128
_T = 4096
_B = 16
_PL0 = 256
_PN0 = 16
_PN1 = 4
_IR3 = 1.0 / math.sqrt(3.0)

_D1 = (_IR3, 0.5, 0.5, _IR3)
_U1 = tuple(
    _D1[j] * ((_D1[j - 1] if j > 0 else 0.0) + 2.0 * _D1[j] + (_D1[j + 1] if j < 3 else 0.0)) / 4.0
    for j in range(4)
)


def _body(len_ref, x_ref, w0_ref, b0_ref, w1_ref, b1_ref, w2_ref, b2_ref, o_ref):
    b = pl.program_id(0)
    ln = len_ref[b]
    xb = x_ref[0]

    j = jax.lax.broadcasted_iota(jnp.int32, (1, _T), 1)
    p = j // _PL0
    jj = j - p * _PL0
    m = jnp.clip(ln - p * _PL0, 0, _PL0)

    half = jnp.float32(0.5)
    ir3 = jnp.float32(_IR3)
    dd = jnp.where((jj == 0) | (jj == _PL0 - 1), ir3, half)
    dm1 = jnp.where(jj == 1, ir3, half)
    dp1 = jnp.where(jj == _PL0 - 2, ir3, half)
    gp = ((jj >= 1) & (jj <= m)).astype(jnp.float32)
    gs = (jj < m).astype(jnp.float32)
    gn = ((jj <= _PL0 - 2) & (jj + 1 < m)).astype(jnp.float32)
    w = dd * (dm1 * gp + 2.0 * dd * gs + dp1 * gn)
    w = w / jnp.maximum(m.astype(jnp.float32), 1.0)

    xw = xb * w
    cols = [
        jnp.sum(xw[:, k * _PL0:(k + 1) * _PL0], axis=1, keepdims=True)
        for k in range(_PN0)
    ]
    s0 = jnp.concatenate(cols, axis=1)

    h0 = jax.lax.dot_general(
        w0_ref[...], s0, (((0,), (0,)), ((), ())),
        preferred_element_type=jnp.float32)
    pidx = jax.lax.broadcasted_iota(jnp.int32, (1, _PN0), 1)
    gate = (ln > pidx * _PL0).astype(jnp.float32)
    h0 = h0 + b0_ref[...] * gate

    cols1 = [
        _U1[0] * h0[:, 4 * q:4 * q + 1]
        + _U1[1] * h0[:, 4 * q + 1:4 * q + 2]
        + _U1[2] * h0[:, 4 * q + 2:4 * q + 3]
        + _U1[3] * h0[:, 4 * q + 3:4 * q + 4]
        for q in range(_PN1)
    ]
    s1 = jnp.concatenate(cols1, axis=1)

    h1 = jax.lax.dot_general(
        w1_ref[...], s1, (((0,), (0,)), ((), ())),
        preferred_element_type=jnp.float32) + b1_ref[...]
    out = jax.lax.dot_general(
        w2_ref[...], h1, (((0,), (0,)), ((), ())),
        preferred_element_type=jnp.float32) + b2_ref[...]
    o_ref[0] = out


def kernel(x, lengths, W0, b0, W1, b1, W2, b2):
    b0c = b0.reshape(_HD, 1)
    b1c = b1.reshape(_HD, 1)
    b2c = b2.reshape(_HD, 1)
    wspec = pl.BlockSpec((_HD, _HD), lambda b, L: (0, 0))
    bspec = pl.BlockSpec((_HD, 1), lambda b, L: (0, 0))
    return pl.pallas_call(
        _body,
        grid_spec=pltpu.PrefetchScalarGridSpec(
            num_scalar_prefetch=1,
            grid=(_B,),
            in_specs=[
                pl.BlockSpec((1, _HD, _T), lambda b, L: (b, 0, 0)),
                wspec, bspec, wspec, bspec, wspec, bspec,
            ],
            out_specs=pl.BlockSpec((1, _HD, _PN1), lambda b, L: (b, 0, 0)),
        ),
        out_shape=jax.ShapeDtypeStruct((_B, _HD, _PN1), jnp.float32),
    )(lengths, x, W0, b0c, W1, b1c, W2, b2c)
